# baseline (speedup 1.0000x reference)
import functools

import jax
import jax.numpy as jnp
from jax import lax
from jax.experimental import pallas as pl
from jax.experimental.pallas import tpu as pltpu
from jax.experimental.pallas import tpu_sc as plsc


def kernel(inputs, position_embeddings):
    batch, seq_len, d_model = inputs.shape
    num_workers = 32
    rows_per_w = seq_len // num_workers
    mesh = plsc.VectorSubcoreMesh(core_axis_name="c", subcore_axis_name="s")

    @functools.partial(
        pl.kernel,
        mesh=mesh,
        out_type=jax.ShapeDtypeStruct((batch * seq_len, d_model), jnp.float32),
        scratch_types=[pltpu.SemaphoreType.DMA],
    )
    def sc_copy(table_hbm, out_hbm, sem):
        wid = lax.axis_index("s") * 2 + lax.axis_index("c")
        base = wid * rows_per_w
        copies = [
            pltpu.async_copy(
                table_hbm.at[pl.ds(base, rows_per_w)],
                out_hbm.at[pl.ds(b * seq_len + base, rows_per_w)],
                sem,
            )
            for b in range(batch)
        ]
        for c in copies:
            c.wait()

    out = sc_copy(position_embeddings)
    return out.reshape(batch, seq_len, d_model)


# speedup vs baseline: 44.7541x; 44.7541x over previous
import functools

import jax
import jax.numpy as jnp
from jax import lax
from jax.experimental import pallas as pl
from jax.experimental.pallas import tpu as pltpu
from jax.experimental.pallas import tpu_sc as plsc


def kernel(inputs, position_embeddings):
    batch, seq_len, d_model = inputs.shape
    num_workers = 32
    rows_per_w = seq_len // num_workers
    mesh = plsc.VectorSubcoreMesh(core_axis_name="c", subcore_axis_name="s")

    chunk = 32
    n_chunks = rows_per_w // chunk

    @functools.partial(
        pl.kernel,
        mesh=mesh,
        out_type=jax.ShapeDtypeStruct((batch * seq_len, d_model), jnp.float32),
        scratch_types=[
            pltpu.VMEM((chunk, d_model), jnp.float32),
            pltpu.VMEM((chunk, d_model), jnp.float32),
            pltpu.SemaphoreType.DMA,
            pltpu.SemaphoreType.DMA,
            pltpu.SemaphoreType.DMA,
            pltpu.SemaphoreType.DMA,
        ],
    )
    def sc_copy(table_hbm, out_hbm, buf0, buf1, rsem0, rsem1, wsem0, wsem1):
        wid = lax.axis_index("s") * 2 + lax.axis_index("c")
        base = wid * rows_per_w
        bufs = (buf0, buf1)
        rsems = (rsem0, rsem1)
        wsems = (wsem0, wsem1)
        reads = [
            pltpu.async_copy(
                table_hbm.at[pl.ds(base + c * chunk, chunk)], bufs[c % 2], rsems[c % 2]
            )
            if c < 2
            else None
            for c in range(n_chunks)
        ]
        for c in range(n_chunks):
            reads[c].wait()
            writes = [
                pltpu.async_copy(
                    bufs[c % 2],
                    out_hbm.at[pl.ds(b * seq_len + base + c * chunk, chunk)],
                    wsems[c % 2],
                )
                for b in range(batch)
            ]
            if c + 2 < n_chunks:
                for w in writes:
                    w.wait()
                reads[c + 2] = pltpu.async_copy(
                    table_hbm.at[pl.ds(base + (c + 2) * chunk, chunk)],
                    bufs[c % 2],
                    rsems[c % 2],
                )
            else:
                for w in writes:
                    w.wait()

    out = sc_copy(position_embeddings)
    return out.reshape(batch, seq_len, d_model)
